# Initial kernel scaffold; baseline (speedup 1.0000x reference)
import jax
import jax.numpy as jnp
from jax.experimental import pallas as pl


def kernel(t, y, incidence, Wc, W1, b1, W2, b2, W3, b3):
    raise NotImplementedError("write your pallas kernel here")


# speedup vs baseline: 1.4050x; 1.4050x over previous
import jax
import jax.numpy as jnp
from jax.experimental import pallas as pl

_EPS = 1e-12


def _pass_a_kernel(y_ref, h_ref, msg_ref, de_ref, dvis_ref):
    i = pl.program_id(0)
    h = h_ref[...]
    dv = jnp.sum(h, axis=1, keepdims=True)
    dvis = jax.lax.rsqrt(jnp.clip(dv, _EPS, None))
    dvis_ref[...] = dvis
    q = jax.nn.softmax(y_ref[...], axis=-1)
    xn = q * dvis
    pmsg = jax.lax.dot_general(h, xn, (((0,), (0,)), ((), ())),
                               preferred_element_type=jnp.float32)
    pde = jnp.sum(h, axis=0, keepdims=True)

    @pl.when(i == 0)
    def _init():
        msg_ref[...] = pmsg
        de_ref[...] = pde

    @pl.when(i != 0)
    def _acc():
        msg_ref[...] += pmsg
        de_ref[...] += pde


def _pass_b_kernel(y_ref, h_ref, dvis_ref, msg_ref, de_ref, wc_ref,
                   w1q_ref, w1o_ref, b1_ref, w2_ref, b2_ref, w3_ref, b3_ref,
                   out_ref):
    de = jnp.clip(de_ref[...], _EPS, None)
    msgn = msg_ref[...] * (1.0 / de).T
    agg = jnp.dot(h_ref[...], msgn,
                  preferred_element_type=jnp.float32)
    q = jax.nn.softmax(y_ref[...], axis=-1)
    obs = jnp.dot(agg * dvis_ref[...], wc_ref[...],
                  preferred_element_type=jnp.float32)
    pre1 = (jnp.dot(q, w1q_ref[...], preferred_element_type=jnp.float32)
            + jnp.dot(obs, w1o_ref[...], preferred_element_type=jnp.float32)
            + b1_ref[...])
    h1 = jnp.tanh(pre1)
    h2 = jnp.tanh(jnp.dot(h1, w2_ref[...],
                          preferred_element_type=jnp.float32) + b2_ref[...])
    log_p = jnp.dot(h2, w3_ref[...],
                    preferred_element_type=jnp.float32) + b3_ref[...]
    log_q = jnp.log(jnp.clip(q, _EPS, None))
    drift = log_p - log_q
    out_ref[...] = drift - jnp.mean(drift, axis=-1, keepdims=True)


def _row_block(n):
    for nb in (2000, 1024, 1000, 512, 500, 256, 250, 200, 128, 125, 100, 8):
        if n % nb == 0 and nb % 8 == 0:
            return nb
    return n


def kernel(t, y, incidence, Wc, W1, b1, W2, b2, W3, b3):
    del t
    n, K = y.shape
    m = incidence.shape[1]
    obs_dim = Wc.shape[1]
    width = W1.shape[0]
    nb = _row_block(n)
    grid = (n // nb,)

    msg, de, dvis = pl.pallas_call(
        _pass_a_kernel,
        grid=grid,
        in_specs=[
            pl.BlockSpec((nb, K), lambda i: (i, 0)),
            pl.BlockSpec((nb, m), lambda i: (i, 0)),
        ],
        out_specs=[
            pl.BlockSpec((m, K), lambda i: (0, 0)),
            pl.BlockSpec((1, m), lambda i: (0, 0)),
            pl.BlockSpec((nb, 1), lambda i: (i, 0)),
        ],
        out_shape=[
            jax.ShapeDtypeStruct((m, K), jnp.float32),
            jax.ShapeDtypeStruct((1, m), jnp.float32),
            jax.ShapeDtypeStruct((n, 1), jnp.float32),
        ],
    )(y, incidence)

    w1q = W1[:, :K].T
    w1o = W1[:, K:].T
    w2t = W2.T
    w3t = W3.T
    b1r = b1.reshape(1, width)
    b2r = b2.reshape(1, width)
    b3r = b3.reshape(1, K)

    full = lambda r, c: pl.BlockSpec((r, c), lambda i: (0, 0))
    drift = pl.pallas_call(
        _pass_b_kernel,
        grid=grid,
        in_specs=[
            pl.BlockSpec((nb, K), lambda i: (i, 0)),
            pl.BlockSpec((nb, m), lambda i: (i, 0)),
            pl.BlockSpec((nb, 1), lambda i: (i, 0)),
            full(m, K),
            full(1, m),
            full(K, obs_dim),
            full(K, width),
            full(obs_dim, width),
            full(1, width),
            full(width, width),
            full(1, width),
            full(width, K),
            full(1, K),
        ],
        out_specs=pl.BlockSpec((nb, K), lambda i: (i, 0)),
        out_shape=jax.ShapeDtypeStruct((n, K), jnp.float32),
    )(y, incidence, dvis, msg, de, Wc, w1q, w1o, b1r, w2t, b2r, w3t, b3r)
    return drift
